# Initial kernel scaffold; baseline (speedup 1.0000x reference)
import functools

import jax
import jax.numpy as jnp
from jax import lax
from jax.experimental import pallas as pl
from jax.experimental.pallas import tpu as pltpu
from jax.experimental.pallas import tpu_sc as plsc

_NC = 2
_NS = 16
_LANES = 16


@functools.lru_cache(maxsize=None)
def _make_sc_aggregate(n_nodes, da, n_edges):
    nw = _NC * _NS
    batch = 80
    ew = n_edges // nw
    nb = ew // batch
    assert ew * nw == n_edges and nb * batch == ew
    nr = n_nodes // _NS
    zr = 125
    assert nr % zr == 0 and n_nodes % _NS == 0

    mesh = plsc.VectorSubcoreMesh(core_axis_name="c", subcore_axis_name="s")

    @functools.partial(
        pl.kernel,
        mesh=mesh,
        out_type=jax.ShapeDtypeStruct((_NC, n_nodes, da), jnp.float32),
        scratch_types=[
            pltpu.VMEM((batch,), jnp.int32),
            pltpu.VMEM((batch,), jnp.int32),
            pltpu.VMEM((batch, da), jnp.float32),
            pltpu.VMEM((zr, da), jnp.float32),
            pltpu.VMEM_SHARED((n_nodes, da), jnp.float32),
            pltpu.SemaphoreType.DMA,
        ],
    )
    def sc_aggr(xa_hbm, src_hbm, dst_hbm, out_hbm, src_b, dst_b, rows_v, zb,
                acc, sem):
        cid = lax.axis_index("c")
        sid = lax.axis_index("s")
        wid = cid * _NS + sid

        def zrow(r, carry):
            def zcol(j, c2):
                zb[r, pl.ds(j * _LANES, _LANES)] = jnp.zeros(
                    (_LANES,), jnp.float32)
                return c2
            return lax.fori_loop(0, da // _LANES, zcol, carry)
        lax.fori_loop(0, zr, zrow, 0)
        for k in range(nr // zr):
            pltpu.sync_copy(zb, acc.at[pl.ds(sid * nr + k * zr, zr)])
        plsc.subcore_barrier()

        def body(b, carry):
            base = wid * ew + b * batch
            pltpu.sync_copy(src_hbm.at[pl.ds(base, batch)], src_b)
            pltpu.sync_copy(dst_hbm.at[pl.ds(base, batch)], dst_b)
            pltpu.async_copy(xa_hbm.at[src_b], rows_v, sem).wait()
            pltpu.sync_copy(rows_v, acc.at[dst_b], add=True)
            return carry
        lax.fori_loop(0, nb, body, 0)
        plsc.subcore_barrier()

        pltpu.sync_copy(acc.at[pl.ds(sid * nr, nr)],
                        out_hbm.at[cid, pl.ds(sid * nr, nr)])

    return sc_aggr


def _mlp(v, w1, b1, w2, b2):
    h = jnp.maximum(jnp.dot(v, w1, preferred_element_type=jnp.float32) + b1,
                    0.0)
    return jnp.dot(h, w2, preferred_element_type=jnp.float32) + b2


def _k_colsum(x_ref, p0_ref, p1_ref, w1_ref, b1_ref, w2_ref, b2_ref,
              xsum_ref):
    f = x_ref.shape[1]
    y1 = x_ref[...] + p0_ref[0, :, :f] + p1_ref[0, :, :f]
    x1 = _mlp(y1, w1_ref[...], b1_ref[...], w2_ref[...], b2_ref[...])

    @pl.when(pl.program_id(0) == 0)
    def _():
        xsum_ref[...] = jnp.zeros_like(xsum_ref)

    xsum_ref[...] += jnp.sum(x1, axis=0, keepdims=True)


def _k_chain(n_nodes, f, xsum_ref, p0r_ref, p1r_ref,
             c2w1, c2b1, c2w2, c2b2, c3w1, c3b1, c3w2, c3b2,
             mw1, mb1, mw2, mb2, u1w1, u1b1, u1w2, u1b2,
             u2w1, u2b1, u2w2, u2b2, u3w1a, u3b1, t0_ref, t1_ref):
    s = p0r_ref[0] + p1r_ref[0]
    m = 1.0 + s[:, f:f + 1]
    xp1 = xsum_ref[...] * (1.0 / n_nodes)
    x2 = _mlp(m * xp1, c2w1[...], c2b1[...], c2w2[...], c2b2[...])
    x3 = _mlp(m * x2, c3w1[...], c3b1[...], c3w2[...], c3b2[...])
    xm = _mlp(m * x3, mw1[...], mb1[...], mw2[...], mb2[...])
    u1 = _mlp(m * jnp.concatenate([xm, x3], axis=1),
              u1w1[...], u1b1[...], u1w2[...], u1b2[...])
    u2 = _mlp(m * jnp.concatenate([u1, x2], axis=1),
              u2w1[...], u2b1[...], u2w2[...], u2b2[...])
    t1 = jnp.dot(u2, u3w1a[...], preferred_element_type=jnp.float32)
    t1_ref[...] = t1
    t0_ref[...] = t1 + u3b1[...]


def _k_final(x_ref, p0_ref, p1_ref, t0_ref, t1_ref, w1b_ref, w2_ref, b2_ref,
             out_ref):
    f = x_ref.shape[1]
    y1 = x_ref[...] + p0_ref[0, :, :f] + p1_ref[0, :, :f]
    deg = p0_ref[0, :, f:f + 1] + p1_ref[0, :, f:f + 1]
    z = jnp.dot(y1, w1b_ref[...], preferred_element_type=jnp.float32)
    h = jnp.maximum(t0_ref[...] + deg * t1_ref[...] + z, 0.0)
    out_ref[...] = (jnp.dot(h, w2_ref[...], preferred_element_type=jnp.float32)
                    + b2_ref[...])


def _row(b):
    return b.reshape(1, -1)


def kernel(x, edge_index, params):
    n, f = x.shape
    e = edge_index.shape[1]
    da = f + 16
    src = edge_index[0]
    dst = edge_index[1]
    xa = jnp.concatenate(
        [x, jnp.ones((n, 1), x.dtype), jnp.zeros((n, da - f - 1), x.dtype)],
        axis=1)

    parts = _make_sc_aggregate(n, da, e)(xa, src, dst)

    r = 1000
    g = n // r
    full = lambda i: (0, 0)
    c1w1, c1b1, c1w2, c1b2 = params['c1']
    xsum = pl.pallas_call(
        _k_colsum,
        grid=(g,),
        in_specs=[
            pl.BlockSpec((r, f), lambda i: (i, 0)),
            pl.BlockSpec((1, r, da), lambda i: (0, i, 0)),
            pl.BlockSpec((1, r, da), lambda i: (1, i, 0)),
            pl.BlockSpec((f, 64), full),
            pl.BlockSpec((1, 64), full),
            pl.BlockSpec((64, 64), full),
            pl.BlockSpec((1, 64), full),
        ],
        out_specs=pl.BlockSpec((1, 64), full),
        out_shape=jax.ShapeDtypeStruct((1, 64), jnp.float32),
    )(x, parts, parts, c1w1, _row(c1b1), c1w2, _row(c1b2))

    u3w1, u3b1, u3w2, u3b2 = params['u3']
    chain_ins = []
    for name in ('c2', 'c3', 'mid', 'u1', 'u2'):
        w1, b1, w2, b2 = params[name]
        chain_ins += [w1, _row(b1), w2, _row(b2)]
    nofull = lambda: (0, 0)
    t0, t1 = pl.pallas_call(
        functools.partial(_k_chain, n, f),
        in_specs=[
            pl.BlockSpec((1, 64), nofull),
            pl.BlockSpec((1, 1, da), lambda: (0, 0, 0)),
            pl.BlockSpec((1, 1, da), lambda: (1, 0, 0)),
        ] + [pl.BlockSpec(a.shape, nofull) for a in chain_ins]
        + [pl.BlockSpec((64, 64), nofull), pl.BlockSpec((1, 64), nofull)],
        out_specs=[pl.BlockSpec((1, 64), nofull),
                   pl.BlockSpec((1, 64), nofull)],
        out_shape=[jax.ShapeDtypeStruct((1, 64), jnp.float32),
                   jax.ShapeDtypeStruct((1, 64), jnp.float32)],
    )(xsum, parts, parts, *chain_ins, u3w1[:64], _row(u3b1))

    out = pl.pallas_call(
        _k_final,
        grid=(g,),
        in_specs=[
            pl.BlockSpec((r, f), lambda i: (i, 0)),
            pl.BlockSpec((1, r, da), lambda i: (0, i, 0)),
            pl.BlockSpec((1, r, da), lambda i: (1, i, 0)),
            pl.BlockSpec((1, 64), full),
            pl.BlockSpec((1, 64), full),
            pl.BlockSpec((f, 64), full),
            pl.BlockSpec((64, 64), full),
            pl.BlockSpec((1, 64), full),
        ],
        out_specs=pl.BlockSpec((r, 64), lambda i: (i, 0)),
        out_shape=jax.ShapeDtypeStruct((n, 64), jnp.float32),
    )(x, parts, parts, t0, t1, u3w1[64:], u3w2, _row(u3b2))
    return out


# speedup vs baseline: 21.9813x; 21.9813x over previous
import functools

import jax
import jax.numpy as jnp
from jax import lax
from jax.experimental import pallas as pl
from jax.experimental.pallas import tpu as pltpu
from jax.experimental.pallas import tpu_sc as plsc

_NC = 2
_NS = 16
_LANES = 16


@functools.lru_cache(maxsize=None)
def _make_sc_aggregate(n_nodes, da, n_edges):
    nw = _NC * _NS
    batch = 80
    ew = n_edges // nw
    nb = ew // batch
    assert ew * nw == n_edges and nb * batch == ew
    nr = n_nodes // _NS
    zr = 125
    assert nr % zr == 0 and n_nodes % _NS == 0

    mesh = plsc.VectorSubcoreMesh(core_axis_name="c", subcore_axis_name="s")

    @functools.partial(
        pl.kernel,
        mesh=mesh,
        compiler_params=pltpu.CompilerParams(use_tc_tiling_on_sc=False),
        out_type=jax.ShapeDtypeStruct((_NC, n_nodes, da), jnp.float32),
        scratch_types=[
            pltpu.VMEM((batch,), jnp.int32),
            pltpu.VMEM((batch,), jnp.int32),
            pltpu.VMEM((batch, da), jnp.float32),
            pltpu.VMEM((zr, da), jnp.float32),
            pltpu.VMEM_SHARED((n_nodes, da), jnp.float32),
            pltpu.SemaphoreType.DMA,
        ],
    )
    def sc_aggr(xa_hbm, src_hbm, dst_hbm, out_hbm, src_b, dst_b, rows_v, zb,
                acc, sem):
        cid = lax.axis_index("c")
        sid = lax.axis_index("s")
        wid = cid * _NS + sid

        def zrow(r, carry):
            def zcol(j, c2):
                zb[r, pl.ds(j * _LANES, _LANES)] = jnp.zeros(
                    (_LANES,), jnp.float32)
                return c2
            return lax.fori_loop(0, da // _LANES, zcol, carry)
        lax.fori_loop(0, zr, zrow, 0)
        for k in range(nr // zr):
            pltpu.sync_copy(zb, acc.at[pl.ds(sid * nr + k * zr, zr)])
        plsc.subcore_barrier()

        def body(b, carry):
            base = wid * ew + b * batch
            pltpu.sync_copy(src_hbm.at[pl.ds(base, batch)], src_b)
            pltpu.sync_copy(dst_hbm.at[pl.ds(base, batch)], dst_b)
            pltpu.async_copy(xa_hbm.at[src_b], rows_v, sem).wait()
            pltpu.sync_copy(rows_v, acc.at[dst_b], add=True)
            return carry
        lax.fori_loop(0, nb, body, 0)
        plsc.subcore_barrier()

        pltpu.sync_copy(acc.at[pl.ds(sid * nr, nr)],
                        out_hbm.at[cid, pl.ds(sid * nr, nr)])

    return sc_aggr


def _mlp(v, w1, b1, w2, b2):
    h = jnp.maximum(jnp.dot(v, w1, preferred_element_type=jnp.float32) + b1,
                    0.0)
    return jnp.dot(h, w2, preferred_element_type=jnp.float32) + b2


def _k_colsum(x_ref, p0_ref, p1_ref, w1_ref, b1_ref, w2_ref, b2_ref,
              xsum_ref):
    f = x_ref.shape[1]
    y1 = x_ref[...] + p0_ref[0, :, :f] + p1_ref[0, :, :f]
    x1 = _mlp(y1, w1_ref[...], b1_ref[...], w2_ref[...], b2_ref[...])

    @pl.when(pl.program_id(0) == 0)
    def _():
        xsum_ref[...] = jnp.zeros_like(xsum_ref)

    xsum_ref[...] += jnp.sum(x1, axis=0, keepdims=True)


def _k_chain(n_nodes, f, xsum_ref, p0r_ref, p1r_ref,
             c2w1, c2b1, c2w2, c2b2, c3w1, c3b1, c3w2, c3b2,
             mw1, mb1, mw2, mb2, u1w1, u1b1, u1w2, u1b2,
             u2w1, u2b1, u2w2, u2b2, u3w1a, u3b1, t0_ref, t1_ref):
    s = p0r_ref[0, 0:1, :] + p1r_ref[0, 0:1, :]
    m = 1.0 + s[:, f:f + 1]
    xp1 = xsum_ref[...] * (1.0 / n_nodes)
    x2 = _mlp(m * xp1, c2w1[...], c2b1[...], c2w2[...], c2b2[...])
    x3 = _mlp(m * x2, c3w1[...], c3b1[...], c3w2[...], c3b2[...])
    xm = _mlp(m * x3, mw1[...], mb1[...], mw2[...], mb2[...])
    u1 = _mlp(m * jnp.concatenate([xm, x3], axis=1),
              u1w1[...], u1b1[...], u1w2[...], u1b2[...])
    u2 = _mlp(m * jnp.concatenate([u1, x2], axis=1),
              u2w1[...], u2b1[...], u2w2[...], u2b2[...])
    t1 = jnp.dot(u2, u3w1a[...], preferred_element_type=jnp.float32)
    t1_ref[...] = t1
    t0_ref[...] = t1 + u3b1[...]


def _k_final(x_ref, p0_ref, p1_ref, t0_ref, t1_ref, w1b_ref, w2_ref, b2_ref,
             out_ref):
    f = x_ref.shape[1]
    y1 = x_ref[...] + p0_ref[0, :, :f] + p1_ref[0, :, :f]
    deg = p0_ref[0, :, f:f + 1] + p1_ref[0, :, f:f + 1]
    z = jnp.dot(y1, w1b_ref[...], preferred_element_type=jnp.float32)
    h = jnp.maximum(t0_ref[...] + deg * t1_ref[...] + z, 0.0)
    out_ref[...] = (jnp.dot(h, w2_ref[...], preferred_element_type=jnp.float32)
                    + b2_ref[...])


def _row(b):
    return b.reshape(1, -1)


def kernel(x, edge_index, params):
    n, f = x.shape
    e = edge_index.shape[1]
    da = f + 16
    src = edge_index[0]
    dst = edge_index[1]
    xa = jnp.concatenate(
        [x, jnp.ones((n, 1), x.dtype), jnp.zeros((n, da - f - 1), x.dtype)],
        axis=1)

    parts = _make_sc_aggregate(n, da, e)(xa, src, dst)

    r = 1000
    g = n // r
    full = lambda i: (0, 0)
    c1w1, c1b1, c1w2, c1b2 = params['c1']
    xsum = pl.pallas_call(
        _k_colsum,
        grid=(g,),
        in_specs=[
            pl.BlockSpec((r, f), lambda i: (i, 0)),
            pl.BlockSpec((1, r, da), lambda i: (0, i, 0)),
            pl.BlockSpec((1, r, da), lambda i: (1, i, 0)),
            pl.BlockSpec((f, 64), full),
            pl.BlockSpec((1, 64), full),
            pl.BlockSpec((64, 64), full),
            pl.BlockSpec((1, 64), full),
        ],
        out_specs=pl.BlockSpec((1, 64), full),
        out_shape=jax.ShapeDtypeStruct((1, 64), jnp.float32),
    )(x, parts, parts, c1w1, _row(c1b1), c1w2, _row(c1b2))

    u3w1, u3b1, u3w2, u3b2 = params['u3']
    chain_ins = []
    for name in ('c2', 'c3', 'mid', 'u1', 'u2'):
        w1, b1, w2, b2 = params[name]
        chain_ins += [w1, _row(b1), w2, _row(b2)]
    t0, t1 = pl.pallas_call(
        functools.partial(_k_chain, n, f),
        grid=(1,),
        in_specs=[
            pl.BlockSpec((1, 64), full),
            pl.BlockSpec((1, 8, da), lambda i: (0, 0, 0)),
            pl.BlockSpec((1, 8, da), lambda i: (1, 0, 0)),
        ] + [pl.BlockSpec(a.shape, full) for a in chain_ins]
        + [pl.BlockSpec((64, 64), full), pl.BlockSpec((1, 64), full)],
        out_specs=[pl.BlockSpec((1, 64), full),
                   pl.BlockSpec((1, 64), full)],
        out_shape=[jax.ShapeDtypeStruct((1, 64), jnp.float32),
                   jax.ShapeDtypeStruct((1, 64), jnp.float32)],
    )(xsum, parts, parts, *chain_ins, u3w1[:64], _row(u3b1))

    out = pl.pallas_call(
        _k_final,
        grid=(g,),
        in_specs=[
            pl.BlockSpec((r, f), lambda i: (i, 0)),
            pl.BlockSpec((1, r, da), lambda i: (0, i, 0)),
            pl.BlockSpec((1, r, da), lambda i: (1, i, 0)),
            pl.BlockSpec((1, 64), full),
            pl.BlockSpec((1, 64), full),
            pl.BlockSpec((f, 64), full),
            pl.BlockSpec((64, 64), full),
            pl.BlockSpec((1, 64), full),
        ],
        out_specs=pl.BlockSpec((r, 64), lambda i: (i, 0)),
        out_shape=jax.ShapeDtypeStruct((n, 64), jnp.float32),
    )(x, parts, parts, t0, t1, u3w1[64:], u3w2, _row(u3b2))
    return out


# speedup vs baseline: 36.8307x; 1.6755x over previous
import functools

import jax
import jax.numpy as jnp
from jax import lax
from jax.experimental import pallas as pl
from jax.experimental.pallas import tpu as pltpu
from jax.experimental.pallas import tpu_sc as plsc

_NC = 2
_NS = 16
_LANES = 16


@functools.lru_cache(maxsize=None)
def _make_sc_aggregate(n_nodes, da, n_edges):
    nw = _NC * _NS
    batch = 100
    ew = n_edges // nw
    nb = ew // batch
    assert ew * nw == n_edges and nb * batch == ew and nb % 2 == 0
    nr = n_nodes // _NS
    zr = 125
    assert nr % zr == 0 and n_nodes % _NS == 0

    ch = 10
    nch = nb // ch
    assert nch * ch == nb and nch % 2 == 0
    mesh = plsc.VectorSubcoreMesh(core_axis_name="c", subcore_axis_name="s")

    @functools.partial(
        pl.kernel,
        mesh=mesh,
        compiler_params=pltpu.CompilerParams(use_tc_tiling_on_sc=False),
        out_type=jax.ShapeDtypeStruct((_NC, n_nodes, da), jnp.float32),
        scratch_types=[
            pltpu.VMEM((ch, batch), jnp.int32),
            pltpu.VMEM((ch, batch), jnp.int32),
            pltpu.VMEM((ch, batch), jnp.int32),
            pltpu.VMEM((ch, batch), jnp.int32),
            pltpu.VMEM((batch, da), jnp.float32),
            pltpu.VMEM((batch, da), jnp.float32),
            pltpu.VMEM_SHARED((n_nodes, da), jnp.float32),
            pltpu.SemaphoreType.DMA,
            pltpu.SemaphoreType.DMA,
            pltpu.SemaphoreType.DMA,
            pltpu.SemaphoreType.DMA,
        ],
    )
    def sc_aggr(xa_hbm, src_hbm, dst_hbm, out_hbm, src_a, dst_a, src_b, dst_b,
                rows0, rows1, acc, gs0, gs1, isa, isb):
        cid = lax.axis_index("c")
        sid = lax.axis_index("s")
        wid = cid * _NS + sid

        def stage(c, sv, dv, sem):
            pltpu.async_copy(src_hbm.at[wid, pl.ds(c * ch, ch)], sv, sem)
            pltpu.async_copy(dst_hbm.at[wid, pl.ds(c * ch, ch)], dv, sem)

        def wait_stage(sv, dv, sem):
            pltpu.make_async_copy(src_hbm.at[0, pl.ds(0, ch)], sv, sem).wait()
            pltpu.make_async_copy(src_hbm.at[0, pl.ds(0, ch)], dv, sem).wait()

        stage(0, src_a, dst_a, isa)
        stage(1, src_b, dst_b, isb)

        def zrow(r, carry):
            def zcol(j, c2):
                rows0[r, pl.ds(j * _LANES, _LANES)] = jnp.zeros(
                    (_LANES,), jnp.float32)
                rows1[r, pl.ds(j * _LANES, _LANES)] = jnp.zeros(
                    (_LANES,), jnp.float32)
                return c2
            return lax.fori_loop(0, da // _LANES, zcol, carry)
        lax.fori_loop(0, batch, zrow, 0)
        base = sid * nr
        nzfull = nr // batch
        for k in range(nzfull):
            pltpu.sync_copy(rows0 if k % 2 == 0 else rows1,
                            acc.at[pl.ds(base + k * batch, batch)])
        rem = nr - nzfull * batch
        if rem:
            pltpu.sync_copy(rows0.at[pl.ds(0, rem)],
                            acc.at[pl.ds(base + nzfull * batch, rem)])
        plsc.subcore_barrier()

        def wait_gather(buf, sem):
            pltpu.make_async_copy(xa_hbm.at[pl.ds(0, batch)], buf, sem).wait()

        def process_chunk(sv, dv):
            pltpu.async_copy(xa_hbm.at[sv.at[0]], rows0, gs0)
            for k in range(ch // 2):
                b0 = 2 * k
                b1 = b0 + 1
                pltpu.async_copy(xa_hbm.at[sv.at[b1]], rows1, gs1)
                wait_gather(rows0, gs0)
                pltpu.sync_copy(rows0, acc.at[dv.at[b0]], add=True)
                if b1 + 1 < ch:
                    pltpu.async_copy(xa_hbm.at[sv.at[b1 + 1]], rows0, gs0)
                wait_gather(rows1, gs1)
                pltpu.sync_copy(rows1, acc.at[dv.at[b1]], add=True)

        def body(cc, carry):
            c0 = 2 * cc
            wait_stage(src_a, dst_a, isa)
            process_chunk(src_a, dst_a)

            @pl.when(c0 + 2 < nch)
            def _():
                stage(c0 + 2, src_a, dst_a, isa)

            wait_stage(src_b, dst_b, isb)
            process_chunk(src_b, dst_b)

            @pl.when(c0 + 3 < nch)
            def _():
                stage(c0 + 3, src_b, dst_b, isb)
            return carry
        lax.fori_loop(0, nch // 2, body, 0)
        plsc.subcore_barrier()

        pltpu.sync_copy(acc.at[pl.ds(sid * nr, nr)],
                        out_hbm.at[cid, pl.ds(sid * nr, nr)])

    return sc_aggr


def _mlp(v, w1, b1, w2, b2):
    h = jnp.maximum(jnp.dot(v, w1, preferred_element_type=jnp.float32) + b1,
                    0.0)
    return jnp.dot(h, w2, preferred_element_type=jnp.float32) + b2


def _k_colsum(x_ref, p0_ref, p1_ref, w1_ref, b1_ref, w2_ref, b2_ref,
              xsum_ref):
    f = x_ref.shape[1]
    y1 = x_ref[...] + p0_ref[0, :, :f] + p1_ref[0, :, :f]
    x1 = _mlp(y1, w1_ref[...], b1_ref[...], w2_ref[...], b2_ref[...])

    @pl.when(pl.program_id(0) == 0)
    def _():
        xsum_ref[...] = jnp.zeros_like(xsum_ref)

    xsum_ref[...] += jnp.sum(x1, axis=0, keepdims=True)


def _k_chain(n_nodes, f, xsum_ref, p0r_ref, p1r_ref,
             c2w1, c2b1, c2w2, c2b2, c3w1, c3b1, c3w2, c3b2,
             mw1, mb1, mw2, mb2, u1w1, u1b1, u1w2, u1b2,
             u2w1, u2b1, u2w2, u2b2, u3w1a, u3b1, t0_ref, t1_ref):
    s = p0r_ref[0, 0:1, :] + p1r_ref[0, 0:1, :]
    m = 1.0 + s[:, f:f + 1]
    xp1 = xsum_ref[...] * (1.0 / n_nodes)
    x2 = _mlp(m * xp1, c2w1[...], c2b1[...], c2w2[...], c2b2[...])
    x3 = _mlp(m * x2, c3w1[...], c3b1[...], c3w2[...], c3b2[...])
    xm = _mlp(m * x3, mw1[...], mb1[...], mw2[...], mb2[...])
    u1 = _mlp(m * jnp.concatenate([xm, x3], axis=1),
              u1w1[...], u1b1[...], u1w2[...], u1b2[...])
    u2 = _mlp(m * jnp.concatenate([u1, x2], axis=1),
              u2w1[...], u2b1[...], u2w2[...], u2b2[...])
    t1 = jnp.dot(u2, u3w1a[...], preferred_element_type=jnp.float32)
    t1_ref[...] = t1
    t0_ref[...] = t1 + u3b1[...]


def _k_final(x_ref, p0_ref, p1_ref, t0_ref, t1_ref, w1b_ref, w2_ref, b2_ref,
             out_ref):
    f = x_ref.shape[1]
    y1 = x_ref[...] + p0_ref[0, :, :f] + p1_ref[0, :, :f]
    deg = p0_ref[0, :, f:f + 1] + p1_ref[0, :, f:f + 1]
    z = jnp.dot(y1, w1b_ref[...], preferred_element_type=jnp.float32)
    h = jnp.maximum(t0_ref[...] + deg * t1_ref[...] + z, 0.0)
    out_ref[...] = (jnp.dot(h, w2_ref[...], preferred_element_type=jnp.float32)
                    + b2_ref[...])


def _row(b):
    return b.reshape(1, -1)


def kernel(x, edge_index, params):
    n, f = x.shape
    e = edge_index.shape[1]
    da = f + 16
    src = edge_index[0]
    dst = edge_index[1]
    xa = jnp.concatenate(
        [x, jnp.ones((n, 1), x.dtype), jnp.zeros((n, da - f - 1), x.dtype)],
        axis=1)

    nw = _NC * _NS
    batch = 100
    nb = e // (nw * batch)
    src3 = src.reshape(nw, nb, batch)
    dst3 = dst.reshape(nw, nb, batch)
    parts = _make_sc_aggregate(n, da, e)(xa, src3, dst3)

    r = 1000
    g = n // r
    full = lambda i: (0, 0)
    c1w1, c1b1, c1w2, c1b2 = params['c1']
    xsum = pl.pallas_call(
        _k_colsum,
        grid=(g,),
        in_specs=[
            pl.BlockSpec((r, f), lambda i: (i, 0)),
            pl.BlockSpec((1, r, da), lambda i: (0, i, 0)),
            pl.BlockSpec((1, r, da), lambda i: (1, i, 0)),
            pl.BlockSpec((f, 64), full),
            pl.BlockSpec((1, 64), full),
            pl.BlockSpec((64, 64), full),
            pl.BlockSpec((1, 64), full),
        ],
        out_specs=pl.BlockSpec((1, 64), full),
        out_shape=jax.ShapeDtypeStruct((1, 64), jnp.float32),
    )(x, parts, parts, c1w1, _row(c1b1), c1w2, _row(c1b2))

    u3w1, u3b1, u3w2, u3b2 = params['u3']
    chain_ins = []
    for name in ('c2', 'c3', 'mid', 'u1', 'u2'):
        w1, b1, w2, b2 = params[name]
        chain_ins += [w1, _row(b1), w2, _row(b2)]
    t0, t1 = pl.pallas_call(
        functools.partial(_k_chain, n, f),
        grid=(1,),
        in_specs=[
            pl.BlockSpec((1, 64), full),
            pl.BlockSpec((1, 8, da), lambda i: (0, 0, 0)),
            pl.BlockSpec((1, 8, da), lambda i: (1, 0, 0)),
        ] + [pl.BlockSpec(a.shape, full) for a in chain_ins]
        + [pl.BlockSpec((64, 64), full), pl.BlockSpec((1, 64), full)],
        out_specs=[pl.BlockSpec((1, 64), full),
                   pl.BlockSpec((1, 64), full)],
        out_shape=[jax.ShapeDtypeStruct((1, 64), jnp.float32),
                   jax.ShapeDtypeStruct((1, 64), jnp.float32)],
    )(xsum, parts, parts, *chain_ins, u3w1[:64], _row(u3b1))

    out = pl.pallas_call(
        _k_final,
        grid=(g,),
        in_specs=[
            pl.BlockSpec((r, f), lambda i: (i, 0)),
            pl.BlockSpec((1, r, da), lambda i: (0, i, 0)),
            pl.BlockSpec((1, r, da), lambda i: (1, i, 0)),
            pl.BlockSpec((1, 64), full),
            pl.BlockSpec((1, 64), full),
            pl.BlockSpec((f, 64), full),
            pl.BlockSpec((64, 64), full),
            pl.BlockSpec((1, 64), full),
        ],
        out_specs=pl.BlockSpec((r, 64), lambda i: (i, 0)),
        out_shape=jax.ShapeDtypeStruct((n, 64), jnp.float32),
    )(x, parts, parts, t0, t1, u3w1[64:], u3w2, _row(u3b2))
    return out


# speedup vs baseline: 51.5120x; 1.3986x over previous
import functools

import jax
import jax.numpy as jnp
from jax import lax
from jax.experimental import pallas as pl
from jax.experimental.pallas import tpu as pltpu
from jax.experimental.pallas import tpu_sc as plsc

_NC = 2
_NS = 16
_LANES = 16
_DD = 16


@functools.lru_cache(maxsize=None)
def _make_sc_aggregate(n_nodes, f, n_edges):
    nw = _NC * _NS
    ew = n_edges // nw
    cw = 1000
    nch = ew // cw
    assert ew * nw == n_edges and nch * cw == ew and nch % 2 == 0
    bsizes = [128] * 7 + [104]
    boffs = [sum(bsizes[:i]) for i in range(len(bsizes))]
    assert sum(bsizes) == cw and len(bsizes) % 2 == 0
    bmax = max(bsizes)
    nr = n_nodes // _NS
    assert n_nodes % _NS == 0
    zr = 25
    assert nr % zr == 0
    mesh = plsc.VectorSubcoreMesh(core_axis_name="c", subcore_axis_name="s")

    @functools.partial(
        pl.kernel,
        mesh=mesh,
        compiler_params=pltpu.CompilerParams(use_tc_tiling_on_sc=False),
        out_type=[
            jax.ShapeDtypeStruct((_NC, n_nodes, f), jnp.float32),
            jax.ShapeDtypeStruct((_NC, n_nodes, _DD), jnp.float32),
        ],
        scratch_types=[
            pltpu.VMEM((cw,), jnp.int32),
            pltpu.VMEM((cw,), jnp.int32),
            pltpu.VMEM((cw,), jnp.int32),
            pltpu.VMEM((cw,), jnp.int32),
            pltpu.VMEM((bmax, f), jnp.float32),
            pltpu.VMEM((bmax, f), jnp.float32),
            pltpu.VMEM((bmax, _DD), jnp.float32),
            pltpu.VMEM((zr, _DD), jnp.float32),
            pltpu.VMEM_SHARED((n_nodes, f), jnp.float32),
            pltpu.VMEM_SHARED((n_nodes, _DD), jnp.float32),
            pltpu.SemaphoreType.DMA,
            pltpu.SemaphoreType.DMA,
            pltpu.SemaphoreType.DMA,
            pltpu.SemaphoreType.DMA,
            pltpu.SemaphoreType.DMA,
            pltpu.SemaphoreType.DMA,
            pltpu.SemaphoreType.DMA,
        ],
    )
    def sc_aggr(x_hbm, src_hbm, dst_hbm, outx_hbm, outd_hbm,
                src_a, dst_a, src_b, dst_b, rows0, rows1, ones_v, zd,
                accx, accd, gs0, gs1, ss0, ss1, dsem, isa, isb):
        cid = lax.axis_index("c")
        sid = lax.axis_index("s")
        wid = cid * _NS + sid

        def stage(c, sv, dv, sem):
            off = wid * ew + c * cw
            pltpu.async_copy(src_hbm.at[pl.ds(off, cw)], sv, sem)
            pltpu.async_copy(dst_hbm.at[pl.ds(off, cw)], dv, sem)

        def wait_stage(sv, dv, sem):
            pltpu.make_async_copy(src_hbm.at[pl.ds(0, cw)], sv, sem).wait()
            pltpu.make_async_copy(src_hbm.at[pl.ds(0, cw)], dv, sem).wait()

        stage(0, src_a, dst_a, isa)
        stage(1, src_b, dst_b, isb)

        def zrow(r, carry):
            def zcol(j, c2):
                rows0[r, pl.ds(j * _LANES, _LANES)] = jnp.zeros(
                    (_LANES,), jnp.float32)
                rows1[r, pl.ds(j * _LANES, _LANES)] = jnp.zeros(
                    (_LANES,), jnp.float32)
                return c2
            lax.fori_loop(0, f // _LANES, zcol, carry)
            ones_v[r, pl.ds(0, _LANES)] = jnp.ones((_LANES,), jnp.float32)
            return carry
        lax.fori_loop(0, bmax, zrow, 0)

        def zdrow(r, carry):
            zd[r, pl.ds(0, _LANES)] = jnp.zeros((_LANES,), jnp.float32)
            return carry
        lax.fori_loop(0, zr, zdrow, 0)

        base = sid * nr
        nzfull = nr // bmax
        for k in range(nzfull):
            pltpu.sync_copy(rows0 if k % 2 == 0 else rows1,
                            accx.at[pl.ds(base + k * bmax, bmax)])
        rem = nr - nzfull * bmax
        if rem:
            pltpu.sync_copy(rows0.at[pl.ds(0, rem)],
                            accx.at[pl.ds(base + nzfull * bmax, rem)])
        for k in range(nr // zr):
            pltpu.sync_copy(zd, accd.at[pl.ds(base + k * zr, zr)])
        plsc.subcore_barrier()

        def gather(sv, b, buf, sem):
            pltpu.async_copy(
                x_hbm.at[sv.at[pl.ds(boffs[b], bsizes[b])]],
                buf.at[pl.ds(0, bsizes[b])], sem)

        def wait_rows(buf, b, sem):
            pltpu.make_async_copy(x_hbm.at[pl.ds(0, bsizes[b])],
                                  buf.at[pl.ds(0, bsizes[b])], sem).wait()

        def wait_deg(b):
            pltpu.make_async_copy(x_hbm.at[pl.ds(0, bsizes[b])],
                                  ones_v.at[pl.ds(0, bsizes[b])], dsem).wait()

        def process_chunk(sv, dv):
            nb = len(bsizes)
            for k in range(nb // 2):
                b0 = 2 * k
                b1 = b0 + 1
                wait_rows(rows0, b0, gs0)
                didx = dv.at[pl.ds(boffs[b0], bsizes[b0])]
                pltpu.async_copy(rows0.at[pl.ds(0, bsizes[b0])],
                                 accx.at[didx], ss0, add=True)
                pltpu.async_copy(ones_v.at[pl.ds(0, bsizes[b0])],
                                 accd.at[didx], dsem, add=True)
                wait_rows(rows1, b1, gs1)
                if b1 + 1 < nb:
                    wait_rows(rows0, b0, ss0)
                    gather(sv, b1 + 1, rows0, gs0)
                didx1 = dv.at[pl.ds(boffs[b1], bsizes[b1])]
                pltpu.async_copy(rows1.at[pl.ds(0, bsizes[b1])],
                                 accx.at[didx1], ss1, add=True)
                pltpu.async_copy(ones_v.at[pl.ds(0, bsizes[b1])],
                                 accd.at[didx1], dsem, add=True)
                if b1 + 2 < nb:
                    wait_rows(rows1, b1, ss1)
                    gather(sv, b1 + 2, rows1, gs1)
                wait_deg(b0)
                wait_deg(b1)
            wait_rows(rows0, nb - 2, ss0)
            wait_rows(rows1, nb - 1, ss1)

        def body(cc, carry):
            c0 = 2 * cc
            wait_stage(src_a, dst_a, isa)
            gather(src_a, 0, rows0, gs0)
            gather(src_a, 1, rows1, gs1)
            process_chunk(src_a, dst_a)

            @pl.when(c0 + 2 < nch)
            def _():
                stage(c0 + 2, src_a, dst_a, isa)

            wait_stage(src_b, dst_b, isb)
            gather(src_b, 0, rows0, gs0)
            gather(src_b, 1, rows1, gs1)
            process_chunk(src_b, dst_b)

            @pl.when(c0 + 3 < nch)
            def _():
                stage(c0 + 3, src_b, dst_b, isb)
            return carry
        lax.fori_loop(0, nch // 2, body, 0)
        plsc.subcore_barrier()

        pltpu.sync_copy(accx.at[pl.ds(base, nr)],
                        outx_hbm.at[cid, pl.ds(base, nr)])
        pltpu.sync_copy(accd.at[pl.ds(base, nr)],
                        outd_hbm.at[cid, pl.ds(base, nr)])

    return sc_aggr


def _mlp(v, w1, b1, w2, b2):
    h = jnp.maximum(
        jnp.dot(v, w1[...], preferred_element_type=jnp.float32)
        + b1[...][None, :], 0.0)
    return (jnp.dot(h, w2[...], preferred_element_type=jnp.float32)
            + b2[...][None, :])


def _k_dense(n, *refs):
    (x_ref, p0_ref, p1_ref, d0_ref, d1_ref,
     c1w1, c1b1, c1w2, c1b2, c2w1, c2b1, c2w2, c2b2,
     c3w1, c3b1, c3w2, c3b2, mw1, mb1, mw2, mb2,
     u1w1, u1b1, u1w2, u1b2, u2w1, u2b1, u2w2, u2b2,
     u3w1, u3b1, u3w2, u3b2, out_ref) = refs
    y1 = x_ref[...] + p0_ref[0] + p1_ref[0]
    deg = d0_ref[0, :, 0:1] + d1_ref[0, :, 0:1]
    x1 = _mlp(y1, c1w1, c1b1, c1w2, c1b2)
    xp1 = jnp.sum(x1, axis=0, keepdims=True) * (1.0 / n)
    m = 1.0 + deg[0:1, :]
    x2 = _mlp(m * xp1, c2w1, c2b1, c2w2, c2b2)
    x3 = _mlp(m * x2, c3w1, c3b1, c3w2, c3b2)
    xm = _mlp(m * x3, mw1, mb1, mw2, mb2)
    u1 = _mlp(m * jnp.concatenate([xm, x3], axis=1), u1w1, u1b1, u1w2, u1b2)
    u2 = _mlp(m * jnp.concatenate([u1, x2], axis=1), u2w1, u2b1, u2w2, u2b2)
    t1 = jnp.dot(u2, u3w1[0:64, :], preferred_element_type=jnp.float32)
    t0 = t1 + u3b1[...][None, :]
    z = jnp.dot(y1, u3w1[64:, :], preferred_element_type=jnp.float32)
    h = jnp.maximum(t0 + deg * t1 + z, 0.0)
    out_ref[...] = (jnp.dot(h, u3w2[...], preferred_element_type=jnp.float32)
                    + u3b2[...][None, :])


def kernel(x, edge_index, params):
    n, f = x.shape
    e = edge_index.shape[1]
    src = edge_index[0]
    dst = edge_index[1]
    px, pd = _make_sc_aggregate(n, f, e)(x, src, dst)

    flat_w = []
    for name in ('c1', 'c2', 'c3', 'mid', 'u1', 'u2', 'u3'):
        flat_w += list(params[name])
    ins = [x, px, px, pd, pd] + flat_w
    specs = []
    for i, a in enumerate(ins):
        if 1 <= i <= 4:
            idx = 0 if i in (1, 3) else 1
            specs.append(pl.BlockSpec(
                (1,) + a.shape[1:], functools.partial(
                    (lambda c, j: (c, 0, 0)), idx)))
        else:
            specs.append(pl.BlockSpec(
                a.shape, functools.partial(
                    (lambda nd, j: (0,) * nd), a.ndim)))
    out = pl.pallas_call(
        functools.partial(_k_dense, n),
        grid=(1,),
        in_specs=specs,
        out_specs=pl.BlockSpec((n, 64), lambda j: (0, 0)),
        out_shape=jax.ShapeDtypeStruct((n, 64), jnp.float32),
        compiler_params=pltpu.CompilerParams(
            vmem_limit_bytes=100 * 1024 * 1024),
    )(*ins)
    return out


# speedup vs baseline: 54.6334x; 1.0606x over previous
import functools

import jax
import jax.numpy as jnp
from jax import lax
from jax.experimental import pallas as pl
from jax.experimental.pallas import tpu as pltpu
from jax.experimental.pallas import tpu_sc as plsc

_NC = 2
_NS = 16
_LANES = 16
_DD = 16


@functools.lru_cache(maxsize=None)
def _make_sc_aggregate(n_nodes, f, n_edges):
    nw = _NC * _NS
    ew = n_edges // nw
    cw = 1000
    nch = ew // cw
    assert ew * nw == n_edges and nch * cw == ew and nch % 2 == 0
    bsizes = [128] * 7 + [104]
    boffs = [sum(bsizes[:i]) for i in range(len(bsizes))]
    assert sum(bsizes) == cw and len(bsizes) % 2 == 0
    bmax = max(bsizes)
    nr = n_nodes // _NS
    assert n_nodes % _NS == 0
    zr = 25
    assert nr % zr == 0
    mesh = plsc.VectorSubcoreMesh(core_axis_name="c", subcore_axis_name="s")

    @functools.partial(
        pl.kernel,
        mesh=mesh,
        compiler_params=pltpu.CompilerParams(use_tc_tiling_on_sc=False),
        out_type=[
            jax.ShapeDtypeStruct((_NC, n_nodes, f), jnp.float32),
            jax.ShapeDtypeStruct((_NC, n_nodes, _DD), jnp.float32),
        ],
        scratch_types=[
            pltpu.VMEM((cw,), jnp.int32),
            pltpu.VMEM((cw,), jnp.int32),
            pltpu.VMEM((cw,), jnp.int32),
            pltpu.VMEM((cw,), jnp.int32),
            pltpu.VMEM((bmax, f), jnp.float32),
            pltpu.VMEM((bmax, f), jnp.float32),
            pltpu.VMEM((bmax, _DD), jnp.float32),
            pltpu.VMEM((zr, _DD), jnp.float32),
            pltpu.VMEM_SHARED((n_nodes, f), jnp.float32),
            pltpu.VMEM_SHARED((n_nodes, _DD), jnp.float32),
            pltpu.SemaphoreType.DMA,
            pltpu.SemaphoreType.DMA,
            pltpu.SemaphoreType.DMA,
            pltpu.SemaphoreType.DMA,
            pltpu.SemaphoreType.DMA,
            pltpu.SemaphoreType.DMA,
            pltpu.SemaphoreType.DMA,
        ],
    )
    def sc_aggr(x_hbm, edge_hbm, outx_hbm, outd_hbm,
                src_a, dst_a, src_b, dst_b, rows0, rows1, ones_v, zd,
                accx, accd, gs0, gs1, ss0, ss1, dsem, isa, isb):
        cid = lax.axis_index("c")
        sid = lax.axis_index("s")
        wid = cid * _NS + sid

        def stage(c, sv, dv, sem):
            off = wid * ew + c * cw
            pltpu.async_copy(edge_hbm.at[0, pl.ds(off, cw)], sv, sem)
            pltpu.async_copy(edge_hbm.at[1, pl.ds(off, cw)], dv, sem)

        def wait_stage(sv, dv, sem):
            pltpu.make_async_copy(edge_hbm.at[0, pl.ds(0, cw)], sv,
                                  sem).wait()
            pltpu.make_async_copy(edge_hbm.at[0, pl.ds(0, cw)], dv,
                                  sem).wait()

        stage(0, src_a, dst_a, isa)
        stage(1, src_b, dst_b, isb)

        def zrow(r, carry):
            def zcol(j, c2):
                rows0[r, pl.ds(j * _LANES, _LANES)] = jnp.zeros(
                    (_LANES,), jnp.float32)
                rows1[r, pl.ds(j * _LANES, _LANES)] = jnp.zeros(
                    (_LANES,), jnp.float32)
                return c2
            lax.fori_loop(0, f // _LANES, zcol, carry)
            ones_v[r, pl.ds(0, _LANES)] = jnp.ones((_LANES,), jnp.float32)
            return carry
        lax.fori_loop(0, bmax, zrow, 0)

        def zdrow(r, carry):
            zd[r, pl.ds(0, _LANES)] = jnp.zeros((_LANES,), jnp.float32)
            return carry
        lax.fori_loop(0, zr, zdrow, 0)

        base = sid * nr
        nzfull = nr // bmax
        for k in range(nzfull):
            pltpu.sync_copy(rows0 if k % 2 == 0 else rows1,
                            accx.at[pl.ds(base + k * bmax, bmax)])
        rem = nr - nzfull * bmax
        if rem:
            pltpu.sync_copy(rows0.at[pl.ds(0, rem)],
                            accx.at[pl.ds(base + nzfull * bmax, rem)])
        for k in range(nr // zr):
            pltpu.sync_copy(zd, accd.at[pl.ds(base + k * zr, zr)])
        plsc.subcore_barrier()

        def gather(sv, b, buf, sem):
            pltpu.async_copy(
                x_hbm.at[sv.at[pl.ds(boffs[b], bsizes[b])]],
                buf.at[pl.ds(0, bsizes[b])], sem)

        def wait_rows(buf, b, sem):
            pltpu.make_async_copy(x_hbm.at[pl.ds(0, bsizes[b])],
                                  buf.at[pl.ds(0, bsizes[b])], sem).wait()

        def wait_deg(b):
            pltpu.make_async_copy(x_hbm.at[pl.ds(0, bsizes[b])],
                                  ones_v.at[pl.ds(0, bsizes[b])], dsem).wait()

        def process_chunk(sv, dv):
            nb = len(bsizes)
            for k in range(nb // 2):
                b0 = 2 * k
                b1 = b0 + 1
                wait_rows(rows0, b0, gs0)
                didx = dv.at[pl.ds(boffs[b0], bsizes[b0])]
                pltpu.async_copy(rows0.at[pl.ds(0, bsizes[b0])],
                                 accx.at[didx], ss0, add=True)
                pltpu.async_copy(ones_v.at[pl.ds(0, bsizes[b0])],
                                 accd.at[didx], dsem, add=True)
                wait_rows(rows1, b1, gs1)
                if b1 + 1 < nb:
                    wait_rows(rows0, b0, ss0)
                    gather(sv, b1 + 1, rows0, gs0)
                didx1 = dv.at[pl.ds(boffs[b1], bsizes[b1])]
                pltpu.async_copy(rows1.at[pl.ds(0, bsizes[b1])],
                                 accx.at[didx1], ss1, add=True)
                pltpu.async_copy(ones_v.at[pl.ds(0, bsizes[b1])],
                                 accd.at[didx1], dsem, add=True)
                if b1 + 2 < nb:
                    wait_rows(rows1, b1, ss1)
                    gather(sv, b1 + 2, rows1, gs1)
                wait_deg(b0)
                wait_deg(b1)
            wait_rows(rows0, nb - 2, ss0)
            wait_rows(rows1, nb - 1, ss1)

        def body(cc, carry):
            c0 = 2 * cc
            wait_stage(src_a, dst_a, isa)
            gather(src_a, 0, rows0, gs0)
            gather(src_a, 1, rows1, gs1)
            process_chunk(src_a, dst_a)

            @pl.when(c0 + 2 < nch)
            def _():
                stage(c0 + 2, src_a, dst_a, isa)

            wait_stage(src_b, dst_b, isb)
            gather(src_b, 0, rows0, gs0)
            gather(src_b, 1, rows1, gs1)
            process_chunk(src_b, dst_b)

            @pl.when(c0 + 3 < nch)
            def _():
                stage(c0 + 3, src_b, dst_b, isb)
            return carry
        lax.fori_loop(0, nch // 2, body, 0)
        plsc.subcore_barrier()

        pltpu.sync_copy(accx.at[pl.ds(base, nr)],
                        outx_hbm.at[cid, pl.ds(base, nr)])
        pltpu.sync_copy(accd.at[pl.ds(base, nr)],
                        outd_hbm.at[cid, pl.ds(base, nr)])

    return sc_aggr


def _mlp(v, w1, b1, w2, b2):
    h = jnp.maximum(
        jnp.dot(v, w1[...], preferred_element_type=jnp.float32)
        + b1[...][None, :], 0.0)
    return (jnp.dot(h, w2[...], preferred_element_type=jnp.float32)
            + b2[...][None, :])


def _k_dense(n, *refs):
    (x_ref, p0_ref, p1_ref, d0_ref, d1_ref,
     c1w1, c1b1, c1w2, c1b2, c2w1, c2b1, c2w2, c2b2,
     c3w1, c3b1, c3w2, c3b2, mw1, mb1, mw2, mb2,
     u1w1, u1b1, u1w2, u1b2, u2w1, u2b1, u2w2, u2b2,
     u3w1, u3b1, u3w2, u3b2, out_ref) = refs
    y1 = x_ref[...] + p0_ref[0] + p1_ref[0]
    deg = d0_ref[0, :, 0:1] + d1_ref[0, :, 0:1]
    x1 = _mlp(y1, c1w1, c1b1, c1w2, c1b2)
    xp1 = jnp.sum(x1, axis=0, keepdims=True) * (1.0 / n)
    m = 1.0 + deg[0:1, :]
    x2 = _mlp(m * xp1, c2w1, c2b1, c2w2, c2b2)
    x3 = _mlp(m * x2, c3w1, c3b1, c3w2, c3b2)
    xm = _mlp(m * x3, mw1, mb1, mw2, mb2)
    u1 = _mlp(m * jnp.concatenate([xm, x3], axis=1), u1w1, u1b1, u1w2, u1b2)
    u2 = _mlp(m * jnp.concatenate([u1, x2], axis=1), u2w1, u2b1, u2w2, u2b2)
    t1 = jnp.dot(u2, u3w1[0:64, :], preferred_element_type=jnp.float32)
    t0 = t1 + u3b1[...][None, :]
    z = jnp.dot(y1, u3w1[64:, :], preferred_element_type=jnp.float32)
    h = jnp.maximum(t0 + deg * t1 + z, 0.0)
    out_ref[...] = (jnp.dot(h, u3w2[...], preferred_element_type=jnp.float32)
                    + u3b2[...][None, :])


def kernel(x, edge_index, params):
    n, f = x.shape
    e = edge_index.shape[1]
    px, pd = _make_sc_aggregate(n, f, e)(x, edge_index)

    flat_w = []
    for name in ('c1', 'c2', 'c3', 'mid', 'u1', 'u2', 'u3'):
        flat_w += list(params[name])
    ins = [x, px, px, pd, pd] + flat_w
    specs = []
    for i, a in enumerate(ins):
        if 1 <= i <= 4:
            idx = 0 if i in (1, 3) else 1
            specs.append(pl.BlockSpec(
                (1,) + a.shape[1:], functools.partial(
                    (lambda c, j: (c, 0, 0)), idx)))
        else:
            specs.append(pl.BlockSpec(
                a.shape, functools.partial(
                    (lambda nd, j: (0,) * nd), a.ndim)))
    out = pl.pallas_call(
        functools.partial(_k_dense, n),
        grid=(1,),
        in_specs=specs,
        out_specs=pl.BlockSpec((n, 64), lambda j: (0, 0)),
        out_shape=jax.ShapeDtypeStruct((n, 64), jnp.float32),
        compiler_params=pltpu.CompilerParams(
            vmem_limit_bytes=100 * 1024 * 1024),
    )(*ins)
    return out


# speedup vs baseline: 56.4537x; 1.0333x over previous
import functools

import jax
import jax.numpy as jnp
from jax import lax
from jax.experimental import pallas as pl
from jax.experimental.pallas import tpu as pltpu
from jax.experimental.pallas import tpu_sc as plsc

_NC = 2
_NS = 16
_LANES = 16
_DD = 16
_NBUF = 4


@functools.lru_cache(maxsize=None)
def _make_sc_aggregate(n_nodes, f, n_edges):
    fh = f // 2
    ew = n_edges // _NS
    cw = 1000
    nch = ew // cw
    assert ew * _NS == n_edges and nch * cw == ew and nch % 2 == 0
    bsizes = [128] * 7 + [104]
    boffs = [sum(bsizes[:i]) for i in range(len(bsizes))]
    nb = len(bsizes)
    assert sum(bsizes) == cw and nb % _NBUF == 0
    bmax = max(bsizes)
    drain_sz = [bsizes[(b - _NBUF) % nb] for b in range(nb)]
    nr = n_nodes // _NS
    assert n_nodes % _NS == 0
    zr = 25
    assert nr % zr == 0 and cw <= n_nodes
    mesh = plsc.VectorSubcoreMesh(core_axis_name="c", subcore_axis_name="s")

    @functools.partial(
        pl.kernel,
        mesh=mesh,
        compiler_params=pltpu.CompilerParams(use_tc_tiling_on_sc=False),
        out_type=[
            jax.ShapeDtypeStruct((n_nodes, f), jnp.float32),
            jax.ShapeDtypeStruct((_NC, n_nodes, _DD), jnp.float32),
            jax.ShapeDtypeStruct((_NC, n_nodes, f // 2), jnp.float32),
        ],
        scratch_types=[
            pltpu.VMEM((cw,), jnp.int32),
            pltpu.VMEM((cw,), jnp.int32),
            pltpu.VMEM((cw,), jnp.int32),
            pltpu.VMEM((cw,), jnp.int32),
            [pltpu.VMEM((bmax, fh), jnp.float32) for _ in range(_NBUF)],
            pltpu.VMEM((bmax, _DD), jnp.float32),
            pltpu.VMEM((zr, _DD), jnp.float32),
            pltpu.VMEM_SHARED((n_nodes, fh), jnp.float32),
            pltpu.VMEM_SHARED((n_nodes, _DD), jnp.float32),
            [pltpu.SemaphoreType.DMA for _ in range(_NBUF)],
            [pltpu.SemaphoreType.DMA for _ in range(_NBUF)],
            pltpu.SemaphoreType.DMA,
            pltpu.SemaphoreType.DMA,
            pltpu.SemaphoreType.DMA,
        ],
    )
    def sc_aggr(x_hbm, edge_hbm, outx_hbm, outd_hbm, xh_hbm,
                src_a, dst_a, src_b, dst_b, rows, ones_v, zd,
                accx, accd, gs, ss, dsem, isa, isb):
        cid = lax.axis_index("c")
        sid = lax.axis_index("s")
        col0 = cid * fh

        def stage(c, sv, dv, sem):
            off = sid * ew + c * cw
            pltpu.async_copy(edge_hbm.at[0, pl.ds(off, cw)], sv, sem)
            pltpu.async_copy(edge_hbm.at[1, pl.ds(off, cw)], dv, sem)

        def wait_stage(sv, dv, sem):
            pltpu.make_async_copy(edge_hbm.at[0, pl.ds(0, cw)], sv,
                                  sem).wait()
            pltpu.make_async_copy(edge_hbm.at[0, pl.ds(0, cw)], dv,
                                  sem).wait()

        stage(0, src_a, dst_a, isa)
        stage(1, src_b, dst_b, isb)

        base = sid * nr
        nxc = nr // bmax + (1 if nr % bmax else 0)
        for q in range(nxc):
            r0 = q * bmax
            rn = min(bmax, nr - r0)
            pltpu.sync_copy(
                x_hbm.at[pl.ds(base + r0, rn), pl.ds(col0, fh)],
                rows[q % _NBUF].at[pl.ds(0, rn)])
            pltpu.sync_copy(
                rows[q % _NBUF].at[pl.ds(0, rn)],
                xh_hbm.at[cid, pl.ds(base + r0, rn)])

        def zrow(r, carry):
            def zcol(j, c2):
                for q in range(_NBUF):
                    rows[q][r, pl.ds(j * _LANES, _LANES)] = jnp.zeros(
                        (_LANES,), jnp.float32)
                return c2
            lax.fori_loop(0, fh // _LANES, zcol, carry)
            ones_v[r, pl.ds(0, _LANES)] = jnp.ones((_LANES,), jnp.float32)
            return carry
        lax.fori_loop(0, bmax, zrow, 0)

        def zdrow(r, carry):
            zd[r, pl.ds(0, _LANES)] = jnp.zeros((_LANES,), jnp.float32)
            return carry
        lax.fori_loop(0, zr, zdrow, 0)

        nzfull = nr // bmax
        for k in range(nzfull):
            pltpu.sync_copy(rows[k % _NBUF],
                            accx.at[pl.ds(base + k * bmax, bmax)])
        rem = nr - nzfull * bmax
        if rem:
            pltpu.sync_copy(rows[0].at[pl.ds(0, rem)],
                            accx.at[pl.ds(base + nzfull * bmax, rem)])
        for k in range(nr // zr):
            pltpu.sync_copy(zd, accd.at[pl.ds(base + k * zr, zr)])
        plsc.subcore_barrier()

        def process_chunk(sv, dv, mydeg, first, poststage=None):
            def process(k):
                jk = k % _NBUF
                sz = bsizes[k]
                pltpu.make_async_copy(
                    xh_hbm.at[0, pl.ds(0, sz)],
                    rows[jk].at[pl.ds(0, sz)], gs[jk]).wait()
                didx = dv.at[pl.ds(boffs[k], sz)]
                pltpu.async_copy(rows[jk].at[pl.ds(0, sz)],
                                 accx.at[didx], ss[jk], add=True)

                @pl.when(mydeg)
                def _():
                    pltpu.async_copy(ones_v.at[pl.ds(0, sz)],
                                     accd.at[didx], dsem, add=True)

            for b in range(nb):
                jb = b % _NBUF
                if not (first and b < _NBUF):
                    dsz = drain_sz[b]
                    pltpu.make_async_copy(
                        xh_hbm.at[0, pl.ds(0, dsz)],
                        rows[jb].at[pl.ds(0, dsz)], ss[jb]).wait()
                pltpu.async_copy(
                    xh_hbm.at[cid].at[sv.at[pl.ds(boffs[b], bsizes[b])]],
                    rows[jb].at[pl.ds(0, bsizes[b])], gs[jb])
                if b == _NBUF and poststage is not None:
                    poststage()
                if b >= 2:
                    process(b - 2)
            process(nb - 2)
            process(nb - 1)

            @pl.when(mydeg)
            def _():
                pltpu.make_async_copy(outd_hbm.at[0, pl.ds(0, cw)],
                                      accd.at[pl.ds(0, cw)], dsem).wait()

        def body_common(c0, first):
            def stage_b():
                @pl.when(c0 + 1 < nch)
                def _():
                    stage(c0 + 1, src_b, dst_b, isb)

            def stage_a():
                @pl.when(c0 + 2 < nch)
                def _():
                    stage(c0 + 2, src_a, dst_a, isa)

            wait_stage(src_a, dst_a, isa)
            process_chunk(src_a, dst_a, (c0 % 2) == cid, first,
                          None if first else stage_b)
            wait_stage(src_b, dst_b, isb)
            process_chunk(src_b, dst_b, ((c0 + 1) % 2) == cid, False,
                          stage_a)

        body_common(0, True)

        def body(cc, carry):
            body_common(2 * cc, False)
            return carry
        lax.fori_loop(1, nch // 2, body, 0)

        for j in range(_NBUF):
            sz = bsizes[nb - _NBUF + j]
            pltpu.make_async_copy(xh_hbm.at[0, pl.ds(0, sz)],
                                  rows[j].at[pl.ds(0, sz)], ss[j]).wait()
        plsc.subcore_barrier()

        pltpu.sync_copy(
            accx.at[pl.ds(base, nr)],
            outx_hbm.at[pl.ds(base, nr), pl.ds(col0, fh)])
        pltpu.sync_copy(accd.at[pl.ds(base, nr)],
                        outd_hbm.at[cid, pl.ds(base, nr)])

    return sc_aggr


def _mlp(v, w1, b1, w2, b2):
    h = jnp.maximum(
        jnp.dot(v, w1[...], preferred_element_type=jnp.float32)
        + b1[...][None, :], 0.0)
    return (jnp.dot(h, w2[...], preferred_element_type=jnp.float32)
            + b2[...][None, :])


def _k_dense(n, *refs):
    (x_ref, ag_ref, d0_ref, d1_ref,
     c1w1, c1b1, c1w2, c1b2, c2w1, c2b1, c2w2, c2b2,
     c3w1, c3b1, c3w2, c3b2, mw1, mb1, mw2, mb2,
     u1w1, u1b1, u1w2, u1b2, u2w1, u2b1, u2w2, u2b2,
     u3w1, u3b1, u3w2, u3b2, out_ref) = refs
    y1 = x_ref[...] + ag_ref[...]
    deg = d0_ref[0, :, 0:1] + d1_ref[0, :, 0:1]
    x1 = _mlp(y1, c1w1, c1b1, c1w2, c1b2)
    xp1 = jnp.sum(x1, axis=0, keepdims=True) * (1.0 / n)
    m = 1.0 + deg[0:1, :]
    x2 = _mlp(m * xp1, c2w1, c2b1, c2w2, c2b2)
    x3 = _mlp(m * x2, c3w1, c3b1, c3w2, c3b2)
    xm = _mlp(m * x3, mw1, mb1, mw2, mb2)
    u1 = _mlp(m * jnp.concatenate([xm, x3], axis=1), u1w1, u1b1, u1w2, u1b2)
    u2 = _mlp(m * jnp.concatenate([u1, x2], axis=1), u2w1, u2b1, u2w2, u2b2)
    t1 = jnp.dot(u2, u3w1[0:64, :], preferred_element_type=jnp.float32)
    t0 = t1 + u3b1[...][None, :]
    z = jnp.dot(y1, u3w1[64:, :], preferred_element_type=jnp.float32)
    h = jnp.maximum(t0 + deg * t1 + z, 0.0)
    out_ref[...] = (jnp.dot(h, u3w2[...], preferred_element_type=jnp.float32)
                    + u3b2[...][None, :])


def kernel(x, edge_index, params):
    n, f = x.shape
    e = edge_index.shape[1]
    aggr, pd, _ = _make_sc_aggregate(n, f, e)(x, edge_index)

    flat_w = []
    for name in ('c1', 'c2', 'c3', 'mid', 'u1', 'u2', 'u3'):
        flat_w += list(params[name])
    ins = [x, aggr, pd, pd] + flat_w
    specs = []
    for i, a in enumerate(ins):
        if i in (2, 3):
            idx = 0 if i == 2 else 1
            specs.append(pl.BlockSpec(
                (1,) + a.shape[1:], functools.partial(
                    (lambda c, j: (c, 0, 0)), idx)))
        else:
            specs.append(pl.BlockSpec(
                a.shape, functools.partial(
                    (lambda nd, j: (0,) * nd), a.ndim)))
    out = pl.pallas_call(
        functools.partial(_k_dense, n),
        grid=(1,),
        in_specs=specs,
        out_specs=pl.BlockSpec((n, 64), lambda j: (0, 0)),
        out_shape=jax.ShapeDtypeStruct((n, 64), jnp.float32),
        compiler_params=pltpu.CompilerParams(
            vmem_limit_bytes=100 * 1024 * 1024),
    )(*ins)
    return out
